# baseline (device time: 66201 ns/iter reference)
import jax
import jax.numpy as jnp
from jax import lax
from jax.experimental import pallas as pl
from jax.experimental.pallas import tpu as pltpu

N_DEV = 32
LOG2_N = 5


def kernel(x, Wq, K_ext, V_ext, Wo):
    B, Sq, D = x.shape
    _, Skv_l, Hq, Dh = K_ext.shape
    Dq = Wq.shape[1]
    Dout = Wo.shape[1]
    BH = B * Hq

    def body(x_ref, wq_ref, k_ref, v_ref, wo_ref, out_ref,
             acc_num, acc_den, ctx_scr, num_rx, den_rx,
             ss_n, rs_n, ss_d, rs_d):
        my = lax.axis_index("i")

        bsem = pltpu.get_barrier_semaphore()
        for k in range(LOG2_N):
            partner = my ^ (1 << k)
            pl.semaphore_signal(
                bsem, inc=1,
                device_id=(partner,), device_id_type=pl.DeviceIdType.MESH,
            )
        pl.semaphore_wait(bsem, LOG2_N)

        qi = lax.broadcasted_iota(jnp.int32, (Sq, Skv_l), 0)
        kj = lax.broadcasted_iota(jnp.int32, (Sq, Skv_l), 1)
        qb = qi // 64
        kb = kj // 64 + 2 * my
        mask = (qb == kb) | (kb == 0) | ((qb + kb) % 3 == 0)

        for b in range(B):
            q2 = jnp.dot(
                x_ref[b].astype(jnp.bfloat16),
                wq_ref[:, :].astype(jnp.bfloat16),
                preferred_element_type=jnp.float32,
            )
            for h in range(Hq):
                idx = b * Hq + h
                q = q2[:, h * Dh:(h + 1) * Dh].astype(jnp.bfloat16)
                kk = k_ref[b, :, h, :].astype(jnp.bfloat16)
                s = lax.dot_general(
                    q, kk, (((1,), (1,)), ((), ())),
                    preferred_element_type=jnp.float32,
                ) * 0.125
                w = jnp.where(mask, jnp.exp(s), 0.0)
                vv = v_ref[b, :, h, :].astype(jnp.bfloat16)
                acc_num[pl.ds(idx * Sq, Sq), :] = jnp.dot(
                    w.astype(jnp.bfloat16), vv,
                    preferred_element_type=jnp.float32,
                )
                acc_den[:, idx:idx + 1] = w.sum(axis=1, keepdims=True)

        for k in range(LOG2_N):
            partner = my ^ (1 << k)
            rn = pltpu.make_async_remote_copy(
                src_ref=acc_num, dst_ref=num_rx.at[k],
                send_sem=ss_n.at[k], recv_sem=rs_n.at[k],
                device_id=(partner,), device_id_type=pl.DeviceIdType.MESH,
            )
            rd = pltpu.make_async_remote_copy(
                src_ref=acc_den, dst_ref=den_rx.at[k],
                send_sem=ss_d.at[k], recv_sem=rs_d.at[k],
                device_id=(partner,), device_id_type=pl.DeviceIdType.MESH,
            )
            rn.start()
            rd.start()
            rn.wait()
            rd.wait()
            acc_num[:, :] = acc_num[:, :] + num_rx[k]
            acc_den[:, :] = acc_den[:, :] + den_rx[k]

        for b in range(B):
            for h in range(Hq):
                idx = b * Hq + h
                den = acc_den[:, idx:idx + 1]
                ctx_scr[:, h * Dh:(h + 1) * Dh] = (
                    acc_num[pl.ds(idx * Sq, Sq), :] / den
                )
            out_ref[b] = jnp.dot(
                ctx_scr[:, :].astype(jnp.bfloat16),
                wo_ref[:, :].astype(jnp.bfloat16),
                preferred_element_type=jnp.float32,
            )

    return pl.pallas_call(
        body,
        out_shape=jax.ShapeDtypeStruct((B, Sq, Dout), jnp.float32),
        in_specs=[pl.BlockSpec(memory_space=pltpu.VMEM)] * 5,
        out_specs=pl.BlockSpec(memory_space=pltpu.VMEM),
        scratch_shapes=[
            pltpu.VMEM((BH * Sq, Dh), jnp.float32),
            pltpu.VMEM((Sq, BH), jnp.float32),
            pltpu.VMEM((Sq, Hq * Dh), jnp.float32),
            pltpu.VMEM((LOG2_N, BH * Sq, Dh), jnp.float32),
            pltpu.VMEM((LOG2_N, Sq, BH), jnp.float32),
            pltpu.SemaphoreType.DMA((LOG2_N,)),
            pltpu.SemaphoreType.DMA((LOG2_N,)),
            pltpu.SemaphoreType.DMA((LOG2_N,)),
            pltpu.SemaphoreType.DMA((LOG2_N,)),
        ],
        compiler_params=pltpu.CompilerParams(collective_id=0),
    )(x, Wq, K_ext, V_ext, Wo)


# device time: 27875 ns/iter; 2.3749x vs baseline; 2.3749x over previous
import jax
import jax.numpy as jnp
from jax import lax
from jax.experimental import pallas as pl
from jax.experimental.pallas import tpu as pltpu

N_DEV = 32
LOG2_N = 5


def kernel(x, Wq, K_ext, V_ext, Wo):
    B, Sq, D = x.shape
    _, Skv_l, Hq, Dh = K_ext.shape
    Dq = Wq.shape[1]
    Dout = Wo.shape[1]
    BH = B * Hq

    K2 = K_ext.reshape(B, Skv_l, Hq * Dh)
    V2 = V_ext.reshape(B, Skv_l, Hq * Dh)

    def body(x_ref, wq_ref, k_ref, v_ref, wo_ref, out_ref,
             acc0, acc1, accd, ctx_scr, rx0, rx1, rxd,
             ss0, rs0, ss1, rs1, ssd, rsd):
        my = lax.axis_index("i")
        partners = [my ^ (1 << k) for k in range(LOG2_N)]

        bsem = pltpu.get_barrier_semaphore()
        for k in range(LOG2_N):
            pl.semaphore_signal(
                bsem, inc=1,
                device_id=(partners[k],),
                device_id_type=pl.DeviceIdType.MESH,
            )

        qi = lax.broadcasted_iota(jnp.int32, (Sq, Skv_l), 0)
        kj = lax.broadcasted_iota(jnp.int32, (Sq, Skv_l), 1)
        qb = qi // 64
        kb = kj // 64 + 2 * my
        mask = (qb == kb) | (kb == 0) | ((qb + kb) % 3 == 0)

        def partials(b, acc):
            q2 = jnp.dot(
                x_ref[b].astype(jnp.bfloat16),
                wq_ref[:, :].astype(jnp.bfloat16),
                preferred_element_type=jnp.float32,
            )
            for h in range(Hq):
                idx = b * Hq + h
                q = q2[:, h * Dh:(h + 1) * Dh].astype(jnp.bfloat16)
                kk = k_ref[b][:, h * Dh:(h + 1) * Dh].astype(jnp.bfloat16)
                s = lax.dot_general(
                    q, kk, (((1,), (1,)), ((), ())),
                    preferred_element_type=jnp.float32,
                ) * 0.125
                w = jnp.where(mask, jnp.exp(s), 0.0)
                vv = v_ref[b][:, h * Dh:(h + 1) * Dh].astype(jnp.bfloat16)
                acc[:, h * Dh:(h + 1) * Dh] = jnp.dot(
                    w.astype(jnp.bfloat16), vv,
                    preferred_element_type=jnp.float32,
                ).astype(jnp.bfloat16)
                accd[:, idx:idx + 1] = w.sum(
                    axis=1, keepdims=True).astype(jnp.bfloat16)

        def mk(src, dst_arr, k, ss, rs):
            return pltpu.make_async_remote_copy(
                src_ref=src, dst_ref=dst_arr.at[k],
                send_sem=ss.at[k], recv_sem=rs.at[k],
                device_id=(partners[k],),
                device_id_type=pl.DeviceIdType.MESH,
            )

        partials(0, acc0)
        pl.semaphore_wait(bsem, LOG2_N)

        d0 = [None] * LOG2_N
        d1 = [None] * LOG2_N
        dd = [None] * LOG2_N
        d0[0] = mk(acc0, rx0, 0, ss0, rs0)
        d0[0].start()

        partials(1, acc1)
        d1[0] = mk(acc1, rx1, 0, ss1, rs1)
        d1[0].start()
        dd[0] = mk(accd, rxd, 0, ssd, rsd)
        dd[0].start()

        for k in range(LOG2_N):
            d0[k].wait()
            acc0[:, :] = acc0[:, :] + rx0[k]
            if k + 1 < LOG2_N:
                d0[k + 1] = mk(acc0, rx0, k + 1, ss0, rs0)
                d0[k + 1].start()
            d1[k].wait()
            acc1[:, :] = acc1[:, :] + rx1[k]
            if k + 1 < LOG2_N:
                d1[k + 1] = mk(acc1, rx1, k + 1, ss1, rs1)
                d1[k + 1].start()
            dd[k].wait()
            accd[:, :] = accd[:, :] + rxd[k]
            if k + 1 < LOG2_N:
                dd[k + 1] = mk(accd, rxd, k + 1, ssd, rsd)
                dd[k + 1].start()

        for b, acc in ((0, acc0), (1, acc1)):
            for h in range(Hq):
                idx = b * Hq + h
                den = accd[:, idx:idx + 1].astype(jnp.float32)
                ctx_scr[:, h * Dh:(h + 1) * Dh] = (
                    acc[:, h * Dh:(h + 1) * Dh].astype(jnp.float32) / den
                ).astype(jnp.bfloat16)
            out_ref[b] = jnp.dot(
                ctx_scr[:, :],
                wo_ref[:, :].astype(jnp.bfloat16),
                preferred_element_type=jnp.float32,
            )

    return pl.pallas_call(
        body,
        out_shape=jax.ShapeDtypeStruct((B, Sq, Dout), jnp.float32),
        in_specs=[pl.BlockSpec(memory_space=pltpu.VMEM)] * 5,
        out_specs=pl.BlockSpec(memory_space=pltpu.VMEM),
        scratch_shapes=[
            pltpu.VMEM((Sq, Dq), jnp.bfloat16),
            pltpu.VMEM((Sq, Dq), jnp.bfloat16),
            pltpu.VMEM((Sq, BH), jnp.bfloat16),
            pltpu.VMEM((Sq, Hq * Dh), jnp.bfloat16),
            pltpu.VMEM((LOG2_N, Sq, Dq), jnp.bfloat16),
            pltpu.VMEM((LOG2_N, Sq, Dq), jnp.bfloat16),
            pltpu.VMEM((LOG2_N, Sq, BH), jnp.bfloat16),
            pltpu.SemaphoreType.DMA((LOG2_N,)),
            pltpu.SemaphoreType.DMA((LOG2_N,)),
            pltpu.SemaphoreType.DMA((LOG2_N,)),
            pltpu.SemaphoreType.DMA((LOG2_N,)),
            pltpu.SemaphoreType.DMA((LOG2_N,)),
            pltpu.SemaphoreType.DMA((LOG2_N,)),
        ],
        compiler_params=pltpu.CompilerParams(collective_id=0),
    )(x, Wq, K2, V2, Wo)


# device time: 22519 ns/iter; 2.9398x vs baseline; 1.2378x over previous
import jax
import jax.numpy as jnp
from jax import lax
from jax.experimental import pallas as pl
from jax.experimental.pallas import tpu as pltpu

N_DEV = 32
QUAD = (1, 2, 3)
CROSS = (4, 8, 12, 16, 20, 24, 28)
N_SLOTS = len(QUAD) + len(CROSS) + len(QUAD)
_SA, _SB, _SC = 0, 3, 10
QROWS = 32


def kernel(x, Wq, K_ext, V_ext, Wo):
    B, Sq, D = x.shape
    _, Skv_l, Hq, Dh = K_ext.shape
    Dq = Wq.shape[1]
    Dout = Wo.shape[1]
    BH = B * Hq

    K2 = K_ext.reshape(B, Skv_l, Hq * Dh)
    V2 = V_ext.reshape(B, Skv_l, Hq * Dh)

    def body(x_ref, wq_ref, k_ref, v_ref, wo_ref, out_ref,
             acc0, acc1, accd, ctx_scr, rxa0, rxa1, rxad,
             rxb0, rxb1, rxbd,
             ss0, rs0, ss1, rs1, ssd, rsd):
        my = lax.axis_index("i")
        myq = my & 3

        bsem = pltpu.get_barrier_semaphore()
        for d in QUAD + CROSS:
            pl.semaphore_signal(
                bsem, inc=1,
                device_id=(my ^ d,),
                device_id_type=pl.DeviceIdType.MESH,
            )

        qi = lax.broadcasted_iota(jnp.int32, (Sq, Skv_l), 0)
        kj = lax.broadcasted_iota(jnp.int32, (Sq, Skv_l), 1)
        qb = qi // 64
        kb = kj // 64 + 2 * my
        mask = (qb == kb) | (kb == 0) | ((qb + kb) % 3 == 0)

        def partials(b, acc):
            q2 = jnp.dot(
                x_ref[b].astype(jnp.bfloat16),
                wq_ref[:, :].astype(jnp.bfloat16),
                preferred_element_type=jnp.float32,
            )
            for h in range(Hq):
                idx = b * Hq + h
                q = q2[:, h * Dh:(h + 1) * Dh].astype(jnp.bfloat16)
                kk = k_ref[b][:, h * Dh:(h + 1) * Dh].astype(jnp.bfloat16)
                s = lax.dot_general(
                    q, kk, (((1,), (1,)), ((), ())),
                    preferred_element_type=jnp.float32,
                ).astype(jnp.bfloat16) * jnp.bfloat16(0.125)
                w = jnp.where(mask, jnp.exp(s), jnp.bfloat16(0.0))
                vv = v_ref[b][:, h * Dh:(h + 1) * Dh].astype(jnp.bfloat16)
                acc[:, h * Dh:(h + 1) * Dh] = jnp.dot(
                    w, vv,
                    preferred_element_type=jnp.float32,
                ).astype(jnp.bfloat16)
                accd[:, idx:idx + 1] = w.astype(jnp.float32).sum(
                    axis=1, keepdims=True).astype(jnp.bfloat16)

        chains = {
            "p0": (acc0, rxa0, rxb0, ss0, rs0),
            "p1": (acc1, rxa1, rxb1, ss1, rs1),
            "d": (accd, rxad, rxbd, ssd, rsd),
        }
        descs = {name: [None] * N_SLOTS for name in chains}

        def qrows(ref, q):
            return ref.at[pl.ds(q * QROWS, QROWS), :]

        def start_a(name):
            acc, rxa, _, ss, rs = chains[name]
            for j, d in enumerate(QUAD):
                s = _SA + j
                desc = pltpu.make_async_remote_copy(
                    src_ref=qrows(acc, myq ^ d), dst_ref=rxa.at[s - _SA],
                    send_sem=ss.at[s], recv_sem=rs.at[s],
                    device_id=(my ^ d,),
                    device_id_type=pl.DeviceIdType.MESH,
                )
                descs[name][s] = desc
                desc.start()

        def finish_a(name):
            acc, rxa, _, _, _ = chains[name]
            for j in range(len(QUAD)):
                descs[name][_SA + j].wait()
            mine = qrows(acc, myq)
            total = mine[:, :]
            for j in range(len(QUAD)):
                total = total + rxa[j]
            mine[:, :] = total

        def start_b(name):
            acc, _, rxb, ss, rs = chains[name]
            for j, d in enumerate(CROSS):
                s = _SB + j
                desc = pltpu.make_async_remote_copy(
                    src_ref=qrows(acc, myq), dst_ref=rxb.at[s - _SB],
                    send_sem=ss.at[s], recv_sem=rs.at[s],
                    device_id=(my ^ d,),
                    device_id_type=pl.DeviceIdType.MESH,
                )
                descs[name][s] = desc
                desc.start()

        def finish_b(name):
            acc, _, rxb, _, _ = chains[name]
            for j in range(len(CROSS)):
                descs[name][_SB + j].wait()
            mine = qrows(acc, myq)
            total = mine[:, :]
            for j in range(len(CROSS)):
                total = total + rxb[j]
            mine[:, :] = total

        def start_c(name):
            acc, _, _, ss, rs = chains[name]
            for j, d in enumerate(QUAD):
                s = _SC + j
                desc = pltpu.make_async_remote_copy(
                    src_ref=qrows(acc, myq), dst_ref=qrows(acc, myq),
                    send_sem=ss.at[s], recv_sem=rs.at[s],
                    device_id=(my ^ d,),
                    device_id_type=pl.DeviceIdType.MESH,
                )
                descs[name][s] = desc
                desc.start()

        def finish_c(name):
            for j in range(len(QUAD)):
                descs[name][_SC + j].wait()

        partials(0, acc0)
        pl.semaphore_wait(bsem, len(QUAD) + len(CROSS))
        start_a("p0")
        partials(1, acc1)
        start_a("p1")
        start_a("d")

        for name in ("p0", "p1", "d"):
            finish_a(name)
            start_b(name)
        for name in ("p0", "p1", "d"):
            finish_b(name)
            start_c(name)
        for name in ("p0", "p1", "d"):
            finish_c(name)

        for b, acc in ((0, acc0), (1, acc1)):
            for h in range(Hq):
                idx = b * Hq + h
                den = accd[:, idx:idx + 1].astype(jnp.float32)
                ctx_scr[:, h * Dh:(h + 1) * Dh] = (
                    acc[:, h * Dh:(h + 1) * Dh].astype(jnp.float32) / den
                ).astype(jnp.bfloat16)
            out_ref[b] = jnp.dot(
                ctx_scr[:, :],
                wo_ref[:, :].astype(jnp.bfloat16),
                preferred_element_type=jnp.float32,
            )

    return pl.pallas_call(
        body,
        out_shape=jax.ShapeDtypeStruct((B, Sq, Dout), jnp.float32),
        in_specs=[pl.BlockSpec(memory_space=pltpu.VMEM)] * 5,
        out_specs=pl.BlockSpec(memory_space=pltpu.VMEM),
        scratch_shapes=[
            pltpu.VMEM((Sq, Dq), jnp.bfloat16),
            pltpu.VMEM((Sq, Dq), jnp.bfloat16),
            pltpu.VMEM((Sq, BH), jnp.bfloat16),
            pltpu.VMEM((Sq, Hq * Dh), jnp.bfloat16),
            pltpu.VMEM((3, QROWS, Dq), jnp.bfloat16),
            pltpu.VMEM((3, QROWS, Dq), jnp.bfloat16),
            pltpu.VMEM((3, QROWS, BH), jnp.bfloat16),
            pltpu.VMEM((7, QROWS, Dq), jnp.bfloat16),
            pltpu.VMEM((7, QROWS, Dq), jnp.bfloat16),
            pltpu.VMEM((7, QROWS, BH), jnp.bfloat16),
            pltpu.SemaphoreType.DMA((N_SLOTS,)),
            pltpu.SemaphoreType.DMA((N_SLOTS,)),
            pltpu.SemaphoreType.DMA((N_SLOTS,)),
            pltpu.SemaphoreType.DMA((N_SLOTS,)),
            pltpu.SemaphoreType.DMA((N_SLOTS,)),
            pltpu.SemaphoreType.DMA((N_SLOTS,)),
        ],
        compiler_params=pltpu.CompilerParams(collective_id=0),
    )(x, Wq, K2, V2, Wo)
